# baseline (speedup 1.0000x reference)
import functools

import jax
import jax.numpy as jnp
from jax.experimental import pallas as pl
from jax.experimental.pallas import tpu as pltpu


def _fused_kernel(idx_ref, w1_hbm, w2_hbm, out_hbm,
                  hid_ref, w2vm_ref, obuf_ref,
                  sem_g, sem_w, sem_o,
                  *, m, nr, rb):
    c = pl.program_id(0)
    base = c * m

    for b in range(m):
        pltpu.make_async_copy(
            w1_hbm.at[pl.ds(idx_ref[base + b], 1), :],
            hid_ref.at[pl.ds(b, 1), :], sem_g).start()

    pltpu.make_async_copy(w2_hbm, w2vm_ref, sem_w).start(priority=1)

    for b in range(m):
        pltpu.make_async_copy(
            w1_hbm.at[pl.ds(idx_ref[base], 1), :],
            hid_ref.at[pl.ds(0, 1), :], sem_g).wait()
    pltpu.make_async_copy(w2_hbm, w2vm_ref, sem_w).wait()

    for r in range(nr):
        k = r % 2
        rows = pl.ds(r * rb, rb)
        orows = pl.ds(base + r * rb, rb)
        if r >= 2:
            pltpu.make_async_copy(
                obuf_ref.at[k], out_hbm.at[orows, :], sem_o.at[k]).wait()
        obuf_ref[k] = jnp.dot(hid_ref[rows, :], w2vm_ref[...],
                              preferred_element_type=jnp.float32)
        pltpu.make_async_copy(
            obuf_ref.at[k], out_hbm.at[orows, :], sem_o.at[k]).start()

    for r in range(max(nr - 2, 0), nr):
        k = r % 2
        orows = pl.ds(base + r * rb, rb)
        pltpu.make_async_copy(
            obuf_ref.at[k], out_hbm.at[orows, :], sem_o.at[k]).wait()


def kernel(idx, w1, w2):
    (bsz,) = idx.shape
    voc, emb = w1.shape
    assert w2.shape == (emb, voc) and bsz % 2 == 0
    m = bsz // 2
    rb = max(8, m // 4)
    nr = m // rb
    assert nr * rb == m

    grid_spec = pltpu.PrefetchScalarGridSpec(
        num_scalar_prefetch=1,
        grid=(2,),
        in_specs=[
            pl.BlockSpec(memory_space=pl.ANY),
            pl.BlockSpec(memory_space=pl.ANY),
        ],
        out_specs=pl.BlockSpec(memory_space=pl.ANY),
        scratch_shapes=[
            pltpu.VMEM((m, emb), jnp.float32),
            pltpu.VMEM((emb, voc), jnp.float32),
            pltpu.VMEM((2, rb, voc), jnp.float32),
            pltpu.SemaphoreType.DMA,
            pltpu.SemaphoreType.DMA,
            pltpu.SemaphoreType.DMA((2,)),
        ],
    )
    return pl.pallas_call(
        functools.partial(_fused_kernel, m=m, nr=nr, rb=rb),
        grid_spec=grid_spec,
        out_shape=jax.ShapeDtypeStruct((bsz, voc), jnp.float32),
        compiler_params=pltpu.CompilerParams(
            dimension_semantics=("parallel",),
            disable_bounds_checks=True,
        ),
    )(idx, w1, w2)


# speedup vs baseline: 1.3038x; 1.3038x over previous
import functools

import jax
import jax.numpy as jnp
from jax.experimental import pallas as pl
from jax.experimental.pallas import tpu as pltpu


def _fused_kernel(idx_ref, w1_hbm, w2_hbm, out_hbm,
                  hid_ref, w2vm_ref, obuf_ref,
                  sem_g, sem_w, sem_o,
                  *, m, nc, tc):
    c = pl.program_id(0)
    base = c * m

    with jax.named_scope("gather_issue"):
        for b in range(m):
            pltpu.make_async_copy(
                w1_hbm.at[pl.ds(idx_ref[base + b], 1), :],
                hid_ref.at[pl.ds(b, 1), :], sem_g).start()

    with jax.named_scope("w2_issue"):
        for n in range(nc):
            cols = pl.ds(n * tc, tc)
            pltpu.make_async_copy(
                w2_hbm.at[:, cols], w2vm_ref.at[:, cols],
                sem_w.at[n]).start(priority=1)

    with jax.named_scope("gather_wait"):
        for b in range(m):
            pltpu.make_async_copy(
                w1_hbm.at[pl.ds(idx_ref[base], 1), :],
                hid_ref.at[pl.ds(0, 1), :], sem_g).wait()

    for n in range(nc):
        k = n % 2
        cols = pl.ds(n * tc, tc)
        with jax.named_scope(f"w2_wait_{n}"):
            pltpu.make_async_copy(
                w2_hbm.at[:, cols], w2vm_ref.at[:, cols], sem_w.at[n]).wait()
            if n >= 2:
                pltpu.make_async_copy(
                    obuf_ref.at[k], out_hbm.at[pl.ds(base, m), cols],
                    sem_o.at[k]).wait()
        with jax.named_scope(f"dot_{n}"):
            obuf_ref[k] = jnp.dot(hid_ref[...], w2vm_ref[:, cols],
                                  preferred_element_type=jnp.float32)
            pltpu.make_async_copy(
                obuf_ref.at[k], out_hbm.at[pl.ds(base, m), cols],
                sem_o.at[k]).start()

    with jax.named_scope("store_drain"):
        for n in range(max(nc - 2, 0), nc):
            k = n % 2
            cols = pl.ds(n * tc, tc)
            pltpu.make_async_copy(
                obuf_ref.at[k], out_hbm.at[pl.ds(base, m), cols],
                sem_o.at[k]).wait()


def kernel(idx, w1, w2):
    (bsz,) = idx.shape
    voc, emb = w1.shape
    assert w2.shape == (emb, voc) and bsz % 2 == 0
    m = bsz // 2
    tc = min(2048, voc)
    nc = voc // tc
    assert nc * tc == voc

    grid_spec = pltpu.PrefetchScalarGridSpec(
        num_scalar_prefetch=1,
        grid=(2,),
        in_specs=[
            pl.BlockSpec(memory_space=pl.ANY),
            pl.BlockSpec(memory_space=pl.ANY),
        ],
        out_specs=pl.BlockSpec(memory_space=pl.ANY),
        scratch_shapes=[
            pltpu.VMEM((m, emb), jnp.float32),
            pltpu.VMEM((emb, voc), jnp.float32),
            pltpu.VMEM((2, m, tc), jnp.float32),
            pltpu.SemaphoreType.DMA,
            pltpu.SemaphoreType.DMA((nc,)),
            pltpu.SemaphoreType.DMA((2,)),
        ],
    )
    return pl.pallas_call(
        functools.partial(_fused_kernel, m=m, nc=nc, tc=tc),
        grid_spec=grid_spec,
        out_shape=jax.ShapeDtypeStruct((bsz, voc), jnp.float32),
        compiler_params=pltpu.CompilerParams(
            dimension_semantics=("parallel",),
            disable_bounds_checks=True,
        ),
    )(idx, w1, w2)
